# baseline (speedup 1.0000x reference)
import functools

import jax
import jax.numpy as jnp
from jax import lax
from jax.experimental import pallas as pl
from jax.experimental.pallas import tpu as pltpu
from jax.experimental.pallas import tpu_sc as plsc

_LANES = 16
_CHUNK_ROWS = 8
_NBUF = 4
_PREFETCH = 2
_UNROLL = 8


def _make_sc_kernel(batch, seq_len, d_model):
    info = plsc.get_sparse_core_info()
    nc, ns = info.num_cores, info.num_subcores
    nw = nc * ns
    total_rows = batch * seq_len
    rows_per_w = total_rows // nw
    w_per_batch = seq_len // rows_per_w
    c = _CHUNK_ROWS
    vecs_per_row = d_model // _LANES
    nchunks = rows_per_w // c
    mesh = plsc.VectorSubcoreMesh(core_axis_name="c", subcore_axis_name="s")

    buf = lambda: pltpu.VMEM((c, d_model), jnp.float32)
    sem = lambda: pltpu.SemaphoreType.DMA

    @functools.partial(
        pl.kernel,
        mesh=mesh,
        out_type=jax.ShapeDtypeStruct((batch, seq_len, d_model), jnp.float32),
        scratch_types=(
            [buf() for _ in range(_NBUF)]
            + [buf() for _ in range(_NBUF)]
            + [sem() for _ in range(3 * _NBUF)]
        ),
    )
    def sc_add(x_hbm, pos_hbm, out_hbm, *scratch):
        xo_bufs = scratch[0:_NBUF]
        p_bufs = scratch[_NBUF:2 * _NBUF]
        sems = scratch[2 * _NBUF:]
        sx = sems[0:_NBUF]
        sp = sems[_NBUF:2 * _NBUF]
        so = sems[2 * _NBUF:]

        wid = lax.axis_index("s") * nc + lax.axis_index("c")
        b = wid // w_per_batch
        s0 = (wid % w_per_batch) * rows_per_w

        def x_cp(j, u):
            s = s0 + j * c
            return pltpu.make_async_copy(x_hbm.at[b, pl.ds(s, c)], xo_bufs[u], sx[u])

        def p_cp(j, u):
            s = s0 + j * c
            return pltpu.make_async_copy(pos_hbm.at[pl.ds(s, c)], p_bufs[u], sp[u])

        def o_cp(j, u):
            s = s0 + j * c
            return pltpu.make_async_copy(xo_bufs[u], out_hbm.at[b, pl.ds(s, c)], so[u])

        for u in range(_PREFETCH):
            x_cp(u, u).start()
            p_cp(u, u).start()

        def round_body(t, carry):
            for u in range(_NBUF):
                j = t * _NBUF + u
                x_cp(j, u).wait()
                p_cp(j, u).wait()

                xo_v, p_v = xo_bufs[u], p_bufs[u]

                @plsc.parallel_loop(0, c * vecs_per_row, step=1, unroll=_UNROLL)
                def add_body(i):
                    r = i // vecs_per_row
                    k = lax.rem(i, vecs_per_row) * _LANES
                    sl = pl.ds(k, _LANES)
                    plsc.addupdate(xo_v.at[r, sl], p_v[r, sl])

                o_cp(j, u).start()

                jn = j + _PREFETCH
                un = (u + _PREFETCH) % _NBUF

                @pl.when(jn < nchunks)
                def _():
                    @pl.when(jn >= _NBUF)
                    def _():
                        o_cp(jn - _NBUF, un).wait()

                    x_cp(jn, un).start()
                    p_cp(jn, un).start()
            return carry

        lax.fori_loop(0, nchunks // _NBUF, round_body, 0)

        for u in range(_NBUF):
            j = nchunks - _NBUF + u
            o_cp(j, u).wait()

    return sc_add


def kernel(x, pos_weight):
    batch, seq_len, d_model = x.shape
    sc = _make_sc_kernel(batch, seq_len, d_model)
    return sc(x, pos_weight[:seq_len])
